# baseline (speedup 1.0000x reference)
import functools

import jax
import jax.numpy as jnp
from jax.experimental import pallas as pl
from jax.experimental.pallas import tpu as pltpu


def _fused_body(x_ref, w_ref, o_ref, mask_ref):
    @pl.when(pl.program_id(0) == 0)
    def _():
        z = w_ref[...] * 0.5
        zmax = jnp.max(z, axis=-1, keepdims=True)
        lo0 = zmax - 1.0
        hi0 = zmax

        def it(_, c):
            lo, hi = c
            mid = 0.5 * (lo + hi)
            s = jnp.sum(jnp.maximum(z - mid, 0.0) ** 2, axis=-1, keepdims=True)
            ge = s >= 1.0
            return jnp.where(ge, mid, lo), jnp.where(ge, hi, mid)

        lo, hi = jax.lax.fori_loop(0, 50, it, (lo0, hi0))
        tau = 0.5 * (lo + hi)
        mask_ref[...] = jnp.maximum(z - tau, 0.0) ** 2

    xb = x_ref[...]
    for n in range(mask_ref.shape[0]):
        o_ref[:, n, :] = xb * mask_ref[n, :]


@jax.jit
def kernel(x, W):
    B, D = x.shape
    K, _ = W.shape
    BLK = 512
    grid = (B // BLK,)
    return pl.pallas_call(
        _fused_body,
        grid=grid,
        in_specs=[
            pl.BlockSpec((BLK, D), lambda i: (i, 0)),
            pl.BlockSpec((K, D), lambda i: (0, 0)),
        ],
        out_specs=pl.BlockSpec((BLK, K, D), lambda i: (i, 0, 0)),
        out_shape=jax.ShapeDtypeStruct((B, K, D), x.dtype),
        scratch_shapes=[pltpu.VMEM((K, D), jnp.float32)],
    )(x, W)


# speedup vs baseline: 1.0272x; 1.0272x over previous
import functools

import jax
import jax.numpy as jnp
from jax.experimental import pallas as pl
from jax.experimental.pallas import tpu as pltpu


def _fused_body(x_ref, w_ref, o_ref, mask_ref):
    @pl.when(pl.program_id(0) == 0)
    def _():
        z = w_ref[...] * 0.5
        zmax = jnp.max(z, axis=-1, keepdims=True)
        tau0 = zmax - 1.0

        def it(_, tau):
            r = jnp.maximum(z - tau, 0.0)
            g = jnp.sum(r * r, axis=-1, keepdims=True) - 1.0
            dg = 2.0 * jnp.sum(r, axis=-1, keepdims=True)
            return tau + g / dg

        tau = jax.lax.fori_loop(0, 12, it, tau0)
        mask_ref[...] = jnp.maximum(z - tau, 0.0) ** 2

    xb = x_ref[...]
    for n in range(mask_ref.shape[0]):
        o_ref[:, n, :] = xb * mask_ref[n, :]


@jax.jit
def kernel(x, W):
    B, D = x.shape
    K, _ = W.shape
    BLK = 512
    grid = (B // BLK,)
    return pl.pallas_call(
        _fused_body,
        grid=grid,
        in_specs=[
            pl.BlockSpec((BLK, D), lambda i: (i, 0)),
            pl.BlockSpec((K, D), lambda i: (0, 0)),
        ],
        out_specs=pl.BlockSpec((BLK, K, D), lambda i: (i, 0, 0)),
        out_shape=jax.ShapeDtypeStruct((B, K, D), x.dtype),
        scratch_shapes=[pltpu.VMEM((K, D), jnp.float32)],
    )(x, W)


# speedup vs baseline: 1.0558x; 1.0278x over previous
import functools

import jax
import jax.numpy as jnp
from jax.experimental import pallas as pl
from jax.experimental.pallas import tpu as pltpu


def _fused_body(x_ref, w_ref, o_ref, mask_ref):
    @pl.when(pl.program_id(0) == 0)
    def _():
        z = w_ref[...] * 0.5
        zmax = jnp.max(z, axis=-1, keepdims=True)
        tau0 = zmax - 1.0

        def it(_, tau):
            r = jnp.maximum(z - tau, 0.0)
            g = jnp.sum(r * r, axis=-1, keepdims=True) - 1.0
            dg = 2.0 * jnp.sum(r, axis=-1, keepdims=True)
            return tau + g / dg

        tau = jax.lax.fori_loop(0, 10, it, tau0)
        mask_ref[...] = jnp.maximum(z - tau, 0.0) ** 2

    xb = x_ref[...]
    for n in range(mask_ref.shape[0]):
        o_ref[:, n, :] = xb * mask_ref[n, :]


@jax.jit
def kernel(x, W):
    B, D = x.shape
    K, _ = W.shape
    BLK = 1024
    grid = (B // BLK,)
    return pl.pallas_call(
        _fused_body,
        grid=grid,
        in_specs=[
            pl.BlockSpec((BLK, D), lambda i: (i, 0)),
            pl.BlockSpec((K, D), lambda i: (0, 0)),
        ],
        out_specs=pl.BlockSpec((BLK, K, D), lambda i: (i, 0, 0)),
        out_shape=jax.ShapeDtypeStruct((B, K, D), x.dtype),
        scratch_shapes=[pltpu.VMEM((K, D), jnp.float32)],
    )(x, W)
